# baseline (speedup 1.0000x reference)
import functools

import jax
import jax.numpy as jnp
from jax import lax
from jax.experimental import pallas as pl
from jax.experimental.pallas import tpu as pltpu
from jax.experimental.pallas import tpu_sc as plsc

C = 1000
B = 16384
NC, NS, L = 2, 16, 16
NW = NC * NS
ROWS_PER_W = B // NW
ZROWS = 64
ZBUF = ZROWS * C
ZT = ZBUF // NS
NCHUNK = ROWS_PER_W // ZROWS
IDX_W = 128
NIDX = ROWS_PER_W // IDX_W

_mesh = plsc.VectorSubcoreMesh(core_axis_name="c", subcore_axis_name="s")


@functools.partial(
    pl.kernel,
    mesh=_mesh,
    out_type=jax.ShapeDtypeStruct((B * C,), jnp.float32),
    scratch_types=[
        pltpu.VMEM((ROWS_PER_W,), jnp.int32),
        pltpu.VMEM((ZT,), jnp.float32),
        pltpu.VMEM_SHARED((ZBUF,), jnp.float32),
        pltpu.VMEM((NIDX, IDX_W), jnp.int32),
        pltpu.VMEM((NIDX, IDX_W), jnp.float32),
        pltpu.SemaphoreType.DMA,
        pltpu.SemaphoreType.DMA,
    ],
)
def _onehot_sc(y_hbm, out_hbm, idx_v, ztile, zshared, pos_v, ones_v, zsem, ssem):
    sid = lax.axis_index("s")
    wid = sid * NC + lax.axis_index("c")
    base = wid * ROWS_PER_W
    pltpu.sync_copy(y_hbm.at[pl.ds(base, ROWS_PER_W)], idx_v)

    zeros16 = jnp.zeros((L,), jnp.float32)
    ones16 = jnp.ones((L,), jnp.float32)

    assert ZT % (5 * L) == 0
    def zero_body(i, carry):
        for u in range(5):
            ztile[pl.ds((i * 5 + u) * L, L)] = zeros16
        return carry

    lax.fori_loop(0, ZT // (5 * L), zero_body, 0)
    pltpu.sync_copy(ztile, zshared.at[pl.ds(sid * ZT, ZT)])
    plsc.subcore_barrier()

    zcopies = []
    for c in range(NCHUNK):
        dst = out_hbm.at[pl.ds((base + c * ZROWS) * C, ZBUF)]
        zcopies.append(pltpu.async_copy(zshared, dst, zsem))

    iota = lax.iota(jnp.int32, L)
    for j in range(NIDX):
        for k in range(IDX_W // L):
            r = j * IDX_W + k * L
            y16 = idx_v[pl.ds(r, L)]
            pos_v[j, pl.ds(k * L, L)] = (base + r + iota) * C + y16
            ones_v[j, pl.ds(k * L, L)] = ones16

    for cp in zcopies:
        cp.wait()

    scopies = []
    for j in range(NIDX):
        scopies.append(
            pltpu.async_copy(ones_v.at[j], out_hbm.at[pos_v.at[j]], ssem)
        )
    for cp in scopies:
        cp.wait()


def kernel(y_n):
    flat = _onehot_sc(y_n)
    return flat.reshape(B, C)


# speedup vs baseline: 1.8039x; 1.8039x over previous
import functools

import jax
import jax.numpy as jnp
from jax import lax
from jax.experimental import pallas as pl
from jax.experimental.pallas import tpu as pltpu
from jax.experimental.pallas import tpu_sc as plsc

C = 1000
B = 16384
NC, NS, L = 2, 16, 16
NW = NC * NS
ROWS_PER_W = B // NW
CHUNK = 32
NCHUNK = ROWS_PER_W // CHUNK

_mesh = plsc.VectorSubcoreMesh(core_axis_name="c", subcore_axis_name="s")


@functools.partial(
    pl.kernel,
    mesh=_mesh,
    out_type=jax.ShapeDtypeStruct((B, C), jnp.float32),
    scratch_types=[
        pltpu.VMEM((ROWS_PER_W,), jnp.int32),
        pltpu.VMEM((CHUNK, C), jnp.float32),
        pltpu.VMEM((CHUNK, C), jnp.float32),
        pltpu.SemaphoreType.DMA,
        pltpu.SemaphoreType.DMA,
    ],
    compiler_params=pltpu.CompilerParams(needs_layout_passes=False),
)
def _onehot_sc(y_hbm, out_hbm, idx_v, buf0, buf1, sem0, sem1):
    sid = lax.axis_index("s")
    wid = sid * NC + lax.axis_index("c")
    base = wid * ROWS_PER_W
    pltpu.sync_copy(y_hbm.at[pl.ds(base, ROWS_PER_W)], idx_v)

    zeros16 = jnp.zeros((L,), jnp.float32)
    ones16 = jnp.ones((L,), jnp.float32)

    def zero_body(i, carry):
        for r in range(CHUNK):
            buf0[r, pl.ds(i * L, L)] = zeros16
            buf1[r, pl.ds(i * L, L)] = zeros16
        return carry

    lax.fori_loop(0, C // L, zero_body, 0)
    for r in range(CHUNK):
        buf0[r, pl.ds(C - L, L)] = zeros16
        buf1[r, pl.ds(C - L, L)] = zeros16

    iota = lax.iota(jnp.int32, L)
    rows0 = iota
    rows1 = iota + L

    def positions(c):
        y0 = idx_v[pl.ds(c * CHUNK, L)]
        y1 = idx_v[pl.ds(c * CHUNK + L, L)]
        return y0, y1

    bufs = (buf0, buf1)
    sems = (sem0, sem1)
    copies = [None, None]
    for c in range(NCHUNK):
        bsel = c % 2
        buf, sem = bufs[bsel], sems[bsel]
        if c >= 2:
            copies[bsel].wait()
            q0, q1 = positions(c - 2)
            plsc.store_scatter(buf, [rows0, q0], zeros16)
            plsc.store_scatter(buf, [rows1, q1], zeros16)
        p0, p1 = positions(c)
        plsc.store_scatter(buf, [rows0, p0], ones16)
        plsc.store_scatter(buf, [rows1, p1], ones16)
        dst = out_hbm.at[pl.ds(base + c * CHUNK, CHUNK), :]
        copies[bsel] = pltpu.async_copy(buf, dst, sem)
    copies[0].wait()
    copies[1].wait()


def kernel(y_n):
    return _onehot_sc(y_n)
